# Initial kernel scaffold; baseline (speedup 1.0000x reference)
import jax
import jax.numpy as jnp
from jax.experimental import pallas as pl


def kernel(h, x, v, pairlist, d_ij, dir_ij, params):
    raise NotImplementedError("write your pallas kernel here")


# speedup vs baseline: 9.9023x; 9.9023x over previous
import functools
import math

import jax
import jax.numpy as jnp
import numpy as np
from jax import lax
from jax.experimental import pallas as pl
from jax.experimental.pallas import tpu as pltpu
from jax.experimental.pallas import tpu_sc as plsc

N = 10000
E = 160000
A = 128
EB = 64
HEADS = 4
KRBF = 20
EPS = 1e-08
MAXR = 0.5

EP = 163840
N16 = N + 112
NC, NS = 2, 16
NW = NC * NS
EPW = EP // NW
BG = 128
NBLK = EPW // BG
ROWS = N16 // NS
BE = 1024
BN = 1000

_F32 = jnp.float32


def _silu(z):
    return z * jax.nn.sigmoid(z)


def _node_tables(h, wt_i, wt_j):
    def body(h_ref, wi_ref, wj_ref, ti_ref, tj_ref):
        hb = h_ref[...]
        ti_ref[...] = jnp.dot(hb, wi_ref[...], preferred_element_type=_F32)
        tj_ref[...] = jnp.dot(hb, wj_ref[...], preferred_element_type=_F32)

    return pl.pallas_call(
        body,
        grid=(N // BN,),
        in_specs=[
            pl.BlockSpec((BN, A), lambda i: (i, 0)),
            pl.BlockSpec((A, A), lambda i: (0, 0)),
            pl.BlockSpec((A, A), lambda i: (0, 0)),
        ],
        out_specs=[pl.BlockSpec((BN, A), lambda i: (i, 0))] * 2,
        out_shape=[jax.ShapeDtypeStruct((N, A), _F32)] * 2,
    )(h, wt_i, wt_j)


def _sc_gather_pair(ti, tj, ii, jj):
    mesh = plsc.VectorSubcoreMesh(
        core_axis_name="c", subcore_axis_name="s", num_cores=NC, num_subcores=NS)

    @functools.partial(
        pl.kernel,
        out_type=[jax.ShapeDtypeStruct((EP, 128), _F32)] * 2,
        mesh=mesh,
        scratch_types=[
            pltpu.VMEM((BG,), jnp.int32),
            pltpu.VMEM((BG,), jnp.int32),
            pltpu.VMEM((BG, 128), _F32),
            pltpu.VMEM((BG, 128), _F32),
            pltpu.SemaphoreType.DMA,
            pltpu.SemaphoreType.DMA,
        ],
    )
    def k(ti_h, tj_h, ii_h, jj_h, gi_h, gj_h, iv, jv, ri, rj, s1, s2):
        wid = lax.axis_index("s") * NC + lax.axis_index("c")
        base = wid * EPW

        def blk(b, carry):
            off = pl.multiple_of(base + b * BG, BG)
            pltpu.sync_copy(ii_h.at[pl.ds(off, BG)], iv)
            pltpu.sync_copy(jj_h.at[pl.ds(off, BG)], jv)
            ca = pltpu.async_copy(ti_h.at[iv], ri, s1)
            cb = pltpu.async_copy(tj_h.at[jv], rj, s2)
            ca.wait()
            cb.wait()
            pltpu.sync_copy(ri, gi_h.at[pl.ds(off, BG)])
            pltpu.sync_copy(rj, gj_h.at[pl.ds(off, BG)])
            return carry

        lax.fori_loop(0, NBLK, blk, 0)

    return k(ti, tj, ii, jj)


def _edge_mlp(gi, gj, d, w1f, w1d, b1, w2, b2, wa, ba, bein, cen):
    alpha = 0.1
    beta = (2.0 / KRBF * (1.0 - math.exp(-MAXR / alpha))) ** (-2)

    def body(gi_ref, gj_ref, d_ref, w1f_r, w1d_r, b1_r, w2_r, b2_r, wa_r, ba_r,
             bein_r, cen_r, he_o, exc_o):
        gi_b = gi_ref[...]
        gj_b = gj_ref[...]
        db = d_ref[...]
        q = gi_b[:, 64:84] + gj_b[:, 64:84] + bein_r[...]
        rb = jnp.exp(-beta * (jnp.exp(db * (-1.0 / alpha)) - cen_r[...]) ** 2)
        filt = rb * q
        z1 = (gi_b[:, 0:64] + gj_b[:, 0:64]
              + jnp.dot(filt, w1f_r[...], preferred_element_type=_F32)
              + db * w1d_r[...] + b1_r[...])
        h1 = _silu(z1)
        he = jnp.dot(h1, w2_r[...], preferred_element_type=_F32) + b2_r[...]
        aw = jnp.dot(he, wa_r[...], preferred_element_type=_F32) + ba_r[...]
        aw = jnp.where(aw > 0, aw, 2.0 * (jnp.exp(aw * 0.5) - 1.0))
        ex8 = jnp.exp(aw)
        he_o[...] = he
        exc_o[:, 0:8] = ex8
        exc_o[:, 8:128] = jnp.zeros((BE, 120), _F32)

    rep = lambda i: (0, 0)
    return pl.pallas_call(
        body,
        grid=(EP // BE,),
        in_specs=[
            pl.BlockSpec((BE, 128), lambda i: (i, 0)),
            pl.BlockSpec((BE, 128), lambda i: (i, 0)),
            pl.BlockSpec((BE, 1), lambda i: (i, 0)),
            pl.BlockSpec((KRBF, 64), rep),
            pl.BlockSpec((1, 64), rep),
            pl.BlockSpec((1, 64), rep),
            pl.BlockSpec((64, 64), rep),
            pl.BlockSpec((1, 64), rep),
            pl.BlockSpec((64, 8), rep),
            pl.BlockSpec((1, 8), rep),
            pl.BlockSpec((1, KRBF), rep),
            pl.BlockSpec((1, KRBF), rep),
        ],
        out_specs=[
            pl.BlockSpec((BE, 64), lambda i: (i, 0)),
            pl.BlockSpec((BE, 128), lambda i: (i, 0)),
        ],
        out_shape=[
            jax.ShapeDtypeStruct((EP, 64), _F32),
            jax.ShapeDtypeStruct((EP, 128), _F32),
        ],
    )(gi, gj, d, w1f, w1d, b1, w2, b2, wa, ba, bein, cen)


def _sc_scatter16(z128, ii, exc):
    mesh = plsc.VectorSubcoreMesh(
        core_axis_name="c", subcore_axis_name="s", num_cores=1, num_subcores=NS)
    epw = EP // NS
    nblk = epw // BG

    @functools.partial(
        pl.kernel,
        out_type=jax.ShapeDtypeStruct((N16, 128), _F32),
        mesh=mesh,
        scratch_types=[
            pltpu.VMEM((BG,), jnp.int32),
            pltpu.VMEM((BG, 128), _F32),
            pltpu.VMEM_SHARED((N16, 128), _F32),
        ],
    )
    def k(z128_h, ii_h, exc_h, tab_h, iv, pv, shared):
        sid = lax.axis_index("s")
        r0 = pl.multiple_of(sid * ROWS, 8)
        pltpu.sync_copy(z128_h.at[pl.ds(r0, ROWS)], shared.at[pl.ds(r0, ROWS)])
        plsc.subcore_barrier()

        def blk(b, carry):
            off = pl.multiple_of(sid * epw + b * BG, BG)
            pltpu.sync_copy(ii_h.at[pl.ds(off, BG)], iv)
            pltpu.sync_copy(exc_h.at[pl.ds(off, BG)], pv)
            pltpu.sync_copy(pv, shared.at[iv], add=True)
            return carry

        lax.fori_loop(0, nblk, blk, 0)
        plsc.subcore_barrier()
        pltpu.sync_copy(shared.at[pl.ds(r0, ROWS)], tab_h.at[pl.ds(r0, ROWS)])

    return k(z128, ii, exc)


def _sc_gather16(tab, ii):
    mesh = plsc.VectorSubcoreMesh(
        core_axis_name="c", subcore_axis_name="s", num_cores=NC, num_subcores=NS)

    @functools.partial(
        pl.kernel,
        out_type=jax.ShapeDtypeStruct((EP, 128), _F32),
        mesh=mesh,
        scratch_types=[
            pltpu.VMEM((BG,), jnp.int32),
            pltpu.VMEM((BG, 128), _F32),
        ],
    )
    def k(tab_h, ii_h, out_h, iv, rv):
        wid = lax.axis_index("s") * NC + lax.axis_index("c")
        base = wid * EPW

        def blk(b, carry):
            off = pl.multiple_of(base + b * BG, BG)
            pltpu.sync_copy(ii_h.at[pl.ds(off, BG)], iv)
            pltpu.sync_copy(tab_h.at[iv], rv)
            pltpu.sync_copy(rv, out_h.at[pl.ds(off, BG)])
            return carry

        lax.fori_loop(0, NBLK, blk, 0)

    return k(tab, ii)


def _edge_payload(he, exc, denc, dirp, wxp, wv):
    def body(he_ref, exc_ref, denc_ref, dir_ref, wx_r, wv_r, pay_o):
        he_b = he_ref[...]
        att = exc_ref[:, 0:4] / (denc_ref[:, 0:4] + EPS)
        dirb = dir_ref[...]
        cp = jnp.zeros((BE, 64), _F32)
        for hh in range(HEADS):
            cp = cp + att[:, hh:hh + 1] * jnp.dot(
                he_b, wx_r[64 * hh:64 * hh + 64, :], preferred_element_type=_F32)
        co = jnp.tanh(cp)
        al = jnp.dot(co, wv_r[...], preferred_element_type=_F32)
        for hh in range(HEADS):
            pay_o[:, 64 * hh:64 * hh + 64] = he_b * att[:, hh:hh + 1]
        for kk in range(3):
            pay_o[:, 256 + 64 * kk:320 + 64 * kk] = co * dirb[:, kk:kk + 1]
        pay_o[:, 448:512] = jnp.zeros((BE, 64), _F32)
        pay_o[:, 448:456] = al * dirb

    rep = lambda i: (0, 0)
    return pl.pallas_call(
        body,
        grid=(EP // BE,),
        in_specs=[
            pl.BlockSpec((BE, 64), lambda i: (i, 0)),
            pl.BlockSpec((BE, 128), lambda i: (i, 0)),
            pl.BlockSpec((BE, 128), lambda i: (i, 0)),
            pl.BlockSpec((BE, 8), lambda i: (i, 0)),
            pl.BlockSpec((256, 64), rep),
            pl.BlockSpec((64, 1), rep),
        ],
        out_specs=[pl.BlockSpec((BE, 512), lambda i: (i, 0))],
        out_shape=[jax.ShapeDtypeStruct((EP, 512), _F32)],
    )(he, exc, denc, dirp, wxp, wv)[0]


def _sc_scatter512(z128, ii, pay):
    mesh = plsc.VectorSubcoreMesh(
        core_axis_name="c", subcore_axis_name="s", num_cores=NC, num_subcores=NS)
    epw = EP // NS
    nblk = epw // BG

    @functools.partial(
        pl.kernel,
        out_type=jax.ShapeDtypeStruct((N16, 512), _F32),
        mesh=mesh,
        scratch_types=[
            pltpu.VMEM((BG,), jnp.int32),
            pltpu.VMEM((BG, 128), _F32),
            pltpu.VMEM_SHARED((N16, 128), _F32),
        ],
    )
    def k(z128_h, ii_h, pay_h, acc_h, iv, pv, shared):
        cid = lax.axis_index("c")
        sid = lax.axis_index("s")
        r0 = pl.multiple_of(sid * ROWS, 8)

        def do_slice(s):
            pltpu.sync_copy(z128_h.at[pl.ds(r0, ROWS)], shared.at[pl.ds(r0, ROWS)])
            plsc.subcore_barrier()

            def blk(b, carry):
                off = pl.multiple_of(sid * epw + b * BG, BG)
                pltpu.sync_copy(ii_h.at[pl.ds(off, BG)], iv)
                pltpu.sync_copy(pay_h.at[pl.ds(off, BG), pl.ds(128 * s, 128)], pv)
                pltpu.sync_copy(pv, shared.at[iv], add=True)
                return carry

            lax.fori_loop(0, nblk, blk, 0)
            plsc.subcore_barrier()
            pltpu.sync_copy(shared.at[pl.ds(r0, ROWS)],
                            acc_h.at[pl.ds(r0, ROWS), pl.ds(128 * s, 128)])
            plsc.subcore_barrier()

        @pl.when(cid == 0)
        def _():
            do_slice(0)
            do_slice(2)

        @pl.when(cid == 1)
        def _():
            do_slice(1)
            do_slice(3)

    return k(z128, ii, pay)


def _node_finalize(h, x, v, acc, tab, wp1, bp1, wp2, bp2, wn1h, wn1sem, wn1sp,
                   bn1, wn2, bn2, wv1, bv1, wv2):
    def body(h_ref, x_ref, v_ref, acc_ref, tab_ref, wp1_r, bp1_r, wp2_r, bp2_r,
             wn1h_r, wn1sem_r, wn1sp_r, bn1_r, wn2_r, bn2_r, wv1_r, bv1_r,
             wv2_r, hu_o, xu_o, vu_o):
        hb = h_ref[...]
        accb = acc_ref[...]
        cnt = jnp.maximum(tab_ref[:, 4:5], 1.0)
        rcnt = 1.0 / cnt
        nsq = jnp.zeros((BN, 64), _F32)
        for kk in range(3):
            cm = accb[:, 256 + 64 * kk:320 + 64 * kk] * rcnt
            nsq = nsq + cm * cm
        sp = _silu(jnp.dot(_silu(jnp.dot(nsq, wp1_r[...], preferred_element_type=_F32)
                                 + bp1_r[...]),
                           wp2_r[...], preferred_element_type=_F32) + bp2_r[...])
        zin = (jnp.dot(hb, wn1h_r[...], preferred_element_type=_F32)
               + jnp.dot(accb[:, 0:256], wn1sem_r[...], preferred_element_type=_F32)
               + jnp.dot(sp, wn1sp_r[...], preferred_element_type=_F32)
               + bn1_r[...])
        hu = hb + _silu(jnp.dot(_silu(zin), wn2_r[...], preferred_element_type=_F32)
                        + bn2_r[...])
        dv = accb[:, 448:451] * rcnt
        gate = 2.0 * jax.nn.sigmoid(
            jnp.dot(_silu(jnp.dot(hu, wv1_r[...], preferred_element_type=_F32)
                          + bv1_r[...]),
                    wv2_r[...], preferred_element_type=_F32))
        vu = gate * v_ref[...] + dv
        hu_o[...] = hu
        xu_o[...] = x_ref[...] + vu
        vu_o[...] = vu

    rep = lambda i: (0, 0)
    return pl.pallas_call(
        body,
        grid=(N // BN,),
        in_specs=[
            pl.BlockSpec((BN, A), lambda i: (i, 0)),
            pl.BlockSpec((BN, 3), lambda i: (i, 0)),
            pl.BlockSpec((BN, 3), lambda i: (i, 0)),
            pl.BlockSpec((BN, 512), lambda i: (i, 0)),
            pl.BlockSpec((BN, 128), lambda i: (i, 0)),
            pl.BlockSpec((64, 64), rep),
            pl.BlockSpec((1, 64), rep),
            pl.BlockSpec((64, 64), rep),
            pl.BlockSpec((1, 64), rep),
            pl.BlockSpec((A, A), rep),
            pl.BlockSpec((256, A), rep),
            pl.BlockSpec((64, A), rep),
            pl.BlockSpec((1, A), rep),
            pl.BlockSpec((A, A), rep),
            pl.BlockSpec((1, A), rep),
            pl.BlockSpec((A, 64), rep),
            pl.BlockSpec((1, 64), rep),
            pl.BlockSpec((64, 1), rep),
        ],
        out_specs=[
            pl.BlockSpec((BN, A), lambda i: (i, 0)),
            pl.BlockSpec((BN, 3), lambda i: (i, 0)),
            pl.BlockSpec((BN, 3), lambda i: (i, 0)),
        ],
        out_shape=[
            jax.ShapeDtypeStruct((N, A), _F32),
            jax.ShapeDtypeStruct((N, 3), _F32),
            jax.ShapeDtypeStruct((N, 3), _F32),
        ],
    )(h, x, v, acc, tab, wp1, bp1, wp2, bp2, wn1h, wn1sem, wn1sp, bn1, wn2,
      bn2, wv1, bv1, wv2)


def kernel(h, x, v, pairlist, d_ij, dir_ij, params):
    pad = EP - E
    ii = jnp.concatenate([pairlist[0].astype(jnp.int32),
                          jnp.full((pad,), N, jnp.int32)])
    jj = jnp.concatenate([pairlist[1].astype(jnp.int32),
                          jnp.zeros((pad,), jnp.int32)])
    d = jnp.concatenate([d_ij, jnp.zeros((pad,), _F32)])[:, None]
    dirp = jnp.pad(dir_ij, ((0, pad), (0, 5)))

    W_eo1 = params['W_eo1']
    W_ein = params['W_ein']
    wt_i = jnp.pad(jnp.concatenate([W_eo1[:A], W_ein[:A]], axis=1),
                   ((0, 0), (0, 44)))
    wt_j = jnp.pad(jnp.concatenate([W_eo1[A:2 * A], W_ein[A:2 * A]], axis=1),
                   ((0, 0), (0, 44)))
    w1f = W_eo1[2 * A:2 * A + KRBF]
    w1d = W_eo1[2 * A + KRBF][None, :]
    b1 = params['b_eo1'][None, :]
    w2 = params['W_eo2']
    b2 = params['b_eo2'][None, :]
    wa = jnp.pad(params['W_att'], ((0, 0), (0, 4)))
    ba = jnp.pad(params['b_att'], (0, 4))[None, :]
    bein = params['b_ein'][None, :]
    Wx = params['W_xmix']
    wxp = jnp.concatenate([Wx[hh::HEADS] for hh in range(HEADS)], axis=0)
    wv = params['W_vmix']
    Wn1 = params['W_n1']
    s_idx = jnp.arange(256)
    rowp = (s_idx % 64) * HEADS + s_idx // 64
    wn1h = Wn1[:A]
    wn1sem = Wn1[A:A + 256][rowp]
    wn1sp = Wn1[A + 256:]
    bn1 = params['b_n1'][None, :]
    wn2 = params['W_n2']
    bn2 = params['b_n2'][None, :]
    wp1 = params['W_pn1']
    bp1 = params['b_pn1'][None, :]
    wp2 = params['W_pn2']
    bp2 = params['b_pn2'][None, :]
    wv1 = params['W_v1']
    bv1 = params['b_v1'][None, :]
    wv2 = params['W_v2']

    z128 = jnp.zeros((N16, 128), _F32)
    trash = jnp.zeros((N16 - N, A), _F32)

    ti, tj = _node_tables(h, wt_i, wt_j)
    ti = jnp.concatenate([ti, trash])
    tj = jnp.concatenate([tj, trash])
    gi, gj = _sc_gather_pair(ti, tj, ii, jj)
    cen = jnp.asarray(
        np.linspace(math.exp(-MAXR / 0.1), 1.0, KRBF, dtype=np.float32))[None, :]
    he, exc = _edge_mlp(gi, gj, d, w1f, w1d, b1, w2, b2, wa, ba, bein, cen)
    tab = _sc_scatter16(z128, ii, exc)
    denc = _sc_gather16(tab, ii)
    pay = _edge_payload(he, exc, denc, dirp, wxp, wv)
    acc = _sc_scatter512(z128, ii, pay)
    hu, xu, vu = _node_finalize(h, x, v, acc[:N], tab[:N], wp1, bp1, wp2, bp2,
                                wn1h, wn1sem, wn1sp, bn1, wn2, bn2, wv1, bv1,
                                wv2)
    return hu, xu, vu
